# Initial kernel scaffold; baseline (speedup 1.0000x reference)
import jax
import jax.numpy as jnp
from jax.experimental import pallas as pl


def kernel(x, edge_index, W1, b1, g1, be1, W2, b2, g2, be2, W3, b3, g3, be3, W4, b4, g4, be4):
    raise NotImplementedError("write your pallas kernel here")


# speedup vs baseline: 24.3034x; 24.3034x over previous
import functools

import jax
import jax.numpy as jnp
from jax import lax
from jax.experimental import pallas as pl
from jax.experimental.pallas import tpu as pltpu
from jax.experimental.pallas import tpu_sc as plsc

_NC = 2
_NS = 16
_NW = _NC * _NS

_CHUNK_BY_F = {32: 2000, 64: 1000, 128: 400}
_DEG_CHUNK = 2000


def _pad_nodes(n):
    return ((n + _NS * 8 - 1) // (_NS * 8)) * (_NS * 8)


def _sc_mesh():
    return plsc.VectorSubcoreMesh(core_axis_name="c", subcore_axis_name="s",
                                  num_cores=_NC, num_subcores=_NS)


def _degree_partials(dst, n_nodes):
    n_edges = dst.shape[0]
    per_w = n_edges // _NW
    c = _DEG_CHUNK
    n_chunks = per_w // c
    n_pad = _pad_nodes(n_nodes)
    rpt = n_pad // _NS

    @functools.partial(
        pl.kernel,
        out_type=jax.ShapeDtypeStruct((_NC, n_pad, 16), jnp.float32),
        mesh=_sc_mesh(),
        compiler_params=pltpu.CompilerParams(use_tc_tiling_on_sc=False),
        scratch_types=[
            pltpu.VMEM((c,), jnp.int32),
            pltpu.VMEM((c, 16), jnp.float32),
            pltpu.VMEM_SHARED((n_pad, 16), jnp.float32),
        ],
    )
    def deg_kernel(dst_hbm, ones_hbm, z_hbm, out_hbm, dbuf, onesv, acc):
        cid = lax.axis_index("c")
        sid = lax.axis_index("s")
        wid = cid * _NS + sid
        pltpu.sync_copy(ones_hbm, onesv)
        pltpu.sync_copy(z_hbm, acc.at[pl.ds(sid * rpt, rpt)])
        plsc.subcore_barrier()
        base = wid * per_w

        @pl.loop(0, n_chunks)
        def _(j):
            pltpu.sync_copy(dst_hbm.at[pl.ds(base + j * c, c)], dbuf)
            pltpu.sync_copy(onesv, acc.at[dbuf], add=True)

        plsc.subcore_barrier()
        pltpu.sync_copy(acc.at[pl.ds(sid * rpt, rpt)],
                        out_hbm.at[cid, pl.ds(sid * rpt, rpt)])

    ones = jnp.ones((c, 16), jnp.float32)
    zeros = jnp.zeros((rpt, 16), jnp.float32)
    return deg_kernel(dst, ones, zeros)


def _gather_scatter_partials(y, src, dst):
    n_nodes, feat = y.shape
    n_edges = src.shape[0]
    per_w = n_edges // _NW
    c = _CHUNK_BY_F[feat]
    n_chunks = per_w // c
    n_pad = _pad_nodes(n_nodes)
    rpt = n_pad // _NS

    @functools.partial(
        pl.kernel,
        out_type=jax.ShapeDtypeStruct((_NC, n_pad, feat), jnp.float32),
        mesh=_sc_mesh(),
        compiler_params=pltpu.CompilerParams(use_tc_tiling_on_sc=False),
        scratch_types=[
            pltpu.VMEM((c,), jnp.int32),
            pltpu.VMEM((c,), jnp.int32),
            pltpu.VMEM((c, feat), jnp.float32),
            pltpu.VMEM_SHARED((n_pad, feat), jnp.float32),
            pltpu.SemaphoreType.DMA,
        ],
    )
    def gs_kernel(y_hbm, src_hbm, dst_hbm, z_hbm, out_hbm,
                  sbuf, dbuf, rows, acc, sem):
        cid = lax.axis_index("c")
        sid = lax.axis_index("s")
        wid = cid * _NS + sid
        pltpu.sync_copy(z_hbm, acc.at[pl.ds(sid * rpt, rpt)])
        plsc.subcore_barrier()
        base = wid * per_w

        @pl.loop(0, n_chunks)
        def _(j):
            off = base + j * c
            pltpu.sync_copy(src_hbm.at[pl.ds(off, c)], sbuf)
            pltpu.sync_copy(dst_hbm.at[pl.ds(off, c)], dbuf)
            pltpu.async_copy(y_hbm.at[sbuf], rows, sem).wait()
            pltpu.sync_copy(rows, acc.at[dbuf], add=True)

        plsc.subcore_barrier()
        pltpu.sync_copy(acc.at[pl.ds(sid * rpt, rpt)],
                        out_hbm.at[cid, pl.ds(sid * rpt, rpt)])

    zeros = jnp.zeros((rpt, feat), jnp.float32)
    return gs_kernel(y, src, dst, zeros)


def _pre_body(x_ref, w_ref, dp_ref, y_ref, dinv_ref):
    n = x_ref.shape[0]
    deg = dp_ref[0, :n, 0:1] + dp_ref[1, :n, 0:1] + 1.0
    dinv = lax.rsqrt(deg)
    dinv_ref[...] = dinv
    y_ref[...] = jnp.dot(x_ref[...], w_ref[...],
                         preferred_element_type=jnp.float32) * dinv


def _tc_pre(x, w1, deg_parts):
    n, _ = x.shape
    h1 = w1.shape[1]
    return pl.pallas_call(
        _pre_body,
        out_shape=(jax.ShapeDtypeStruct((n, h1), jnp.float32),
                   jax.ShapeDtypeStruct((n, 1), jnp.float32)),
    )(x, w1, deg_parts)


def _norm_block(p_ref, y_ref, dinv_ref, b_ref, g_ref, be_ref, relu_after):
    n = y_ref.shape[0]
    dinv = dinv_ref[...]
    t = (p_ref[0, :n] + p_ref[1, :n] + y_ref[...]) * dinv + b_ref[...]
    t = jnp.maximum(t, 0.0)
    mu = jnp.mean(t, axis=0, keepdims=True)
    var = jnp.mean((t - mu) ** 2, axis=0, keepdims=True)
    h = (t - mu) * lax.rsqrt(var + 1e-5) * g_ref[...] + be_ref[...]
    if relu_after:
        h = jnp.maximum(h, 0.0)
    return h, dinv


def _tc_mid(parts, y, dinv, b, g, be, w_next, relu_after, split_out=False):
    n = y.shape[0]
    f_next = w_next.shape[1]

    def body(p_ref, y_ref, dinv_ref, b_ref, g_ref, be_ref, w_ref, *o_refs):
        h, dv = _norm_block(p_ref, y_ref, dinv_ref, b_ref, g_ref, be_ref,
                            relu_after)
        if split_out:
            half = f_next // 2
            o_refs[0][...] = jnp.dot(h, w_ref[:, :half],
                                     preferred_element_type=jnp.float32) * dv
            o_refs[1][...] = jnp.dot(h, w_ref[:, half:],
                                     preferred_element_type=jnp.float32) * dv
        else:
            o_refs[0][...] = jnp.dot(h, w_ref[...],
                                     preferred_element_type=jnp.float32) * dv

    if split_out:
        out_shape = (jax.ShapeDtypeStruct((n, f_next // 2), jnp.float32),
                     jax.ShapeDtypeStruct((n, f_next // 2), jnp.float32))
    else:
        out_shape = jax.ShapeDtypeStruct((n, f_next), jnp.float32)
    return pl.pallas_call(
        body, out_shape=out_shape,
    )(parts, y, dinv, b[None, :], g[None, :], be[None, :], w_next)


def _tc_final2(parts_a, parts_b, y_a, y_b, dinv, b, g, be):
    n, half = y_a.shape

    def body(pa_ref, pb_ref, ya_ref, yb_ref, dinv_ref, ba_ref, ga_ref,
             bea_ref, bb_ref, gb_ref, beb_ref, o_ref):
        ha, _ = _norm_block(pa_ref, ya_ref, dinv_ref, ba_ref, ga_ref, bea_ref,
                            False)
        hb, _ = _norm_block(pb_ref, yb_ref, dinv_ref, bb_ref, gb_ref, beb_ref,
                            False)
        o_ref[:, :half] = ha
        o_ref[:, half:] = hb

    return pl.pallas_call(
        body, out_shape=jax.ShapeDtypeStruct((n, 2 * half), jnp.float32),
    )(parts_a, parts_b, y_a, y_b, dinv,
      b[None, :half], g[None, :half], be[None, :half],
      b[None, half:], g[None, half:], be[None, half:])


def kernel(x, edge_index, W1, b1, g1, be1, W2, b2, g2, be2,
           W3, b3, g3, be3, W4, b4, g4, be4):
    src = edge_index[0]
    dst = edge_index[1]
    n = x.shape[0]

    deg_parts = _degree_partials(dst, n)
    y1, dinv = _tc_pre(x, W1, deg_parts)

    p1 = _gather_scatter_partials(y1, src, dst)
    y2 = _tc_mid(p1, y1, dinv, b1, g1, be1, W2, relu_after=True)

    p2 = _gather_scatter_partials(y2, src, dst)
    y3 = _tc_mid(p2, y2, dinv, b2, g2, be2, W3, relu_after=False)

    p3 = _gather_scatter_partials(y3, src, dst)
    y4a, y4b = _tc_mid(p3, y3, dinv, b3, g3, be3, W4, relu_after=True,
                       split_out=True)

    p4a = _gather_scatter_partials(y4a, src, dst)
    p4b = _gather_scatter_partials(y4b, src, dst)
    return _tc_final2(p4a, p4b, y4a, y4b, dinv, b4, g4, be4)


# speedup vs baseline: 31.1330x; 1.2810x over previous
import functools

import jax
import jax.numpy as jnp
from jax import lax
from jax.experimental import pallas as pl
from jax.experimental.pallas import tpu as pltpu
from jax.experimental.pallas import tpu_sc as plsc

_NC = 2
_NS = 16
_NW = _NC * _NS

_CHUNK_BY_F = {32: 1000, 64: 400, 128: 400}
_DEG_CHUNK = 2000


def _pad_nodes(n):
    return ((n + _NS * 8 - 1) // (_NS * 8)) * (_NS * 8)


def _sc_mesh():
    return plsc.VectorSubcoreMesh(core_axis_name="c", subcore_axis_name="s",
                                  num_cores=_NC, num_subcores=_NS)


def _degree_partials(dst, n_nodes):
    n_edges = dst.shape[0]
    per_w = n_edges // _NW
    c = _DEG_CHUNK
    n_chunks = per_w // c
    n_pad = _pad_nodes(n_nodes)
    rpt = n_pad // _NS

    @functools.partial(
        pl.kernel,
        out_type=jax.ShapeDtypeStruct((_NC, n_pad, 16), jnp.float32),
        mesh=_sc_mesh(),
        compiler_params=pltpu.CompilerParams(use_tc_tiling_on_sc=False),
        scratch_types=[
            pltpu.VMEM((c,), jnp.int32),
            pltpu.VMEM((c, 16), jnp.float32),
            pltpu.VMEM_SHARED((n_pad, 16), jnp.float32),
        ],
    )
    def deg_kernel(dst_hbm, ones_hbm, z_hbm, out_hbm, dbuf, onesv, acc):
        cid = lax.axis_index("c")
        sid = lax.axis_index("s")
        wid = cid * _NS + sid
        pltpu.sync_copy(ones_hbm, onesv)
        pltpu.sync_copy(z_hbm, acc.at[pl.ds(sid * rpt, rpt)])
        plsc.subcore_barrier()
        base = wid * per_w

        @pl.loop(0, n_chunks)
        def _(j):
            pltpu.sync_copy(dst_hbm.at[pl.ds(base + j * c, c)], dbuf)
            pltpu.sync_copy(onesv, acc.at[dbuf], add=True)

        plsc.subcore_barrier()
        pltpu.sync_copy(acc.at[pl.ds(sid * rpt, rpt)],
                        out_hbm.at[cid, pl.ds(sid * rpt, rpt)])

    ones = jnp.ones((c, 16), jnp.float32)
    zeros = jnp.zeros((rpt, 16), jnp.float32)
    return deg_kernel(dst, ones, zeros)


def _gather_scatter_partials(y, src, dst):
    n_nodes, feat = y.shape
    n_edges = src.shape[0]
    per_w = n_edges // _NW
    c = _CHUNK_BY_F[feat]
    n_chunks = per_w // c
    n_pad = _pad_nodes(n_nodes)
    rpt = n_pad // _NS

    @functools.partial(
        pl.kernel,
        out_type=jax.ShapeDtypeStruct((_NC, n_pad, feat), jnp.float32),
        mesh=_sc_mesh(),
        compiler_params=pltpu.CompilerParams(use_tc_tiling_on_sc=False),
        scratch_types=[
            pltpu.VMEM((per_w,), jnp.int32),
            pltpu.VMEM((per_w,), jnp.int32),
            pltpu.VMEM((c, feat), jnp.float32),
            pltpu.VMEM((c, feat), jnp.float32),
            pltpu.VMEM_SHARED((n_pad, feat), jnp.float32),
            pltpu.SemaphoreType.DMA,
            pltpu.SemaphoreType.DMA,
        ],
    )
    def gs_kernel(y_hbm, src_hbm, dst_hbm, z_hbm, out_hbm,
                  sall, dall, rows0, rows1, acc, sem0, sem1):
        cid = lax.axis_index("c")
        sid = lax.axis_index("s")
        wid = cid * _NS + sid
        pltpu.sync_copy(z_hbm, acc.at[pl.ds(sid * rpt, rpt)])
        base = wid * per_w
        pltpu.sync_copy(src_hbm.at[pl.ds(base, per_w)], sall)
        pltpu.sync_copy(dst_hbm.at[pl.ds(base, per_w)], dall)
        plsc.subcore_barrier()

        def gather(j, rbuf, sem):
            return pltpu.async_copy(
                y_hbm.at[sall.at[pl.ds(j * c, c)]], rbuf, sem)

        gather(0, rows0, sem0)
        gather(1, rows1, sem1)

        @pl.loop(0, n_chunks, step=2)
        def _(j):
            for b, (rbuf, sem) in enumerate(((rows0, sem0), (rows1, sem1))):
                jj = j + b

                @pl.when(jj < n_chunks)
                def _():
                    pltpu.make_async_copy(
                        y_hbm.at[sall.at[pl.ds(jj * c, c)]], rbuf, sem).wait()
                    pltpu.sync_copy(rbuf, acc.at[dall.at[pl.ds(jj * c, c)]],
                                    add=True)

                    @pl.when(jj + 2 < n_chunks)
                    def _():
                        gather(jj + 2, rbuf, sem)

        plsc.subcore_barrier()
        pltpu.sync_copy(acc.at[pl.ds(sid * rpt, rpt)],
                        out_hbm.at[cid, pl.ds(sid * rpt, rpt)])

    zeros = jnp.zeros((rpt, feat), jnp.float32)
    return gs_kernel(y, src, dst, zeros)


def _pre_body(x_ref, w_ref, dp_ref, y_ref, dinv_ref):
    n = x_ref.shape[0]
    deg = dp_ref[0, :n, 0:1] + dp_ref[1, :n, 0:1] + 1.0
    dinv = lax.rsqrt(deg)
    dinv_ref[...] = dinv
    y_ref[...] = jnp.dot(x_ref[...], w_ref[...],
                         preferred_element_type=jnp.float32) * dinv


def _tc_pre(x, w1, deg_parts):
    n, _ = x.shape
    h1 = w1.shape[1]
    return pl.pallas_call(
        _pre_body,
        out_shape=(jax.ShapeDtypeStruct((n, h1), jnp.float32),
                   jax.ShapeDtypeStruct((n, 1), jnp.float32)),
    )(x, w1, deg_parts)


def _norm_block(p_ref, y_ref, dinv_ref, b_ref, g_ref, be_ref, relu_after):
    n = y_ref.shape[0]
    dinv = dinv_ref[...]
    t = (p_ref[0, :n] + p_ref[1, :n] + y_ref[...]) * dinv + b_ref[...]
    t = jnp.maximum(t, 0.0)
    mu = jnp.mean(t, axis=0, keepdims=True)
    var = jnp.mean((t - mu) ** 2, axis=0, keepdims=True)
    h = (t - mu) * lax.rsqrt(var + 1e-5) * g_ref[...] + be_ref[...]
    if relu_after:
        h = jnp.maximum(h, 0.0)
    return h, dinv


def _tc_mid(parts, y, dinv, b, g, be, w_next, relu_after, split_out=False):
    n = y.shape[0]
    f_next = w_next.shape[1]

    def body(p_ref, y_ref, dinv_ref, b_ref, g_ref, be_ref, w_ref, *o_refs):
        h, dv = _norm_block(p_ref, y_ref, dinv_ref, b_ref, g_ref, be_ref,
                            relu_after)
        if split_out:
            half = f_next // 2
            o_refs[0][...] = jnp.dot(h, w_ref[:, :half],
                                     preferred_element_type=jnp.float32) * dv
            o_refs[1][...] = jnp.dot(h, w_ref[:, half:],
                                     preferred_element_type=jnp.float32) * dv
        else:
            o_refs[0][...] = jnp.dot(h, w_ref[...],
                                     preferred_element_type=jnp.float32) * dv

    if split_out:
        out_shape = (jax.ShapeDtypeStruct((n, f_next // 2), jnp.float32),
                     jax.ShapeDtypeStruct((n, f_next // 2), jnp.float32))
    else:
        out_shape = jax.ShapeDtypeStruct((n, f_next), jnp.float32)
    return pl.pallas_call(
        body, out_shape=out_shape,
    )(parts, y, dinv, b[None, :], g[None, :], be[None, :], w_next)


def _tc_final2(parts_a, parts_b, y_a, y_b, dinv, b, g, be):
    n, half = y_a.shape

    def body(pa_ref, pb_ref, ya_ref, yb_ref, dinv_ref, ba_ref, ga_ref,
             bea_ref, bb_ref, gb_ref, beb_ref, o_ref):
        ha, _ = _norm_block(pa_ref, ya_ref, dinv_ref, ba_ref, ga_ref, bea_ref,
                            False)
        hb, _ = _norm_block(pb_ref, yb_ref, dinv_ref, bb_ref, gb_ref, beb_ref,
                            False)
        o_ref[:, :half] = ha
        o_ref[:, half:] = hb

    return pl.pallas_call(
        body, out_shape=jax.ShapeDtypeStruct((n, 2 * half), jnp.float32),
    )(parts_a, parts_b, y_a, y_b, dinv,
      b[None, :half], g[None, :half], be[None, :half],
      b[None, half:], g[None, half:], be[None, half:])


def kernel(x, edge_index, W1, b1, g1, be1, W2, b2, g2, be2,
           W3, b3, g3, be3, W4, b4, g4, be4):
    src = edge_index[0]
    dst = edge_index[1]
    n = x.shape[0]

    deg_parts = _degree_partials(dst, n)
    y1, dinv = _tc_pre(x, W1, deg_parts)

    p1 = _gather_scatter_partials(y1, src, dst)
    y2 = _tc_mid(p1, y1, dinv, b1, g1, be1, W2, relu_after=True)

    p2 = _gather_scatter_partials(y2, src, dst)
    y3 = _tc_mid(p2, y2, dinv, b2, g2, be2, W3, relu_after=False)

    p3 = _gather_scatter_partials(y3, src, dst)
    y4a, y4b = _tc_mid(p3, y3, dinv, b3, g3, be3, W4, relu_after=True,
                       split_out=True)

    p4a = _gather_scatter_partials(y4a, src, dst)
    p4b = _gather_scatter_partials(y4b, src, dst)
    return _tc_final2(p4a, p4b, y4a, y4b, dinv, b4, g4, be4)
